# baseline (speedup 1.0000x reference)
import jax
import jax.numpy as jnp
from jax import lax
from jax.experimental import pallas as pl
from jax.experimental.pallas import tpu as pltpu
from jax.experimental.pallas import tpu_sc as plsc

VOCAB = 1_000_000
MAX_LEN = 200
EMBED_DIM = 64
BATCH = 4096

NC, NS, L = 2, 16, 16
NW = NC * NS
ROWS = BATCH * MAX_LEN
ROWS_PER_W = ROWS // NW
SEQ_PER_CHUNK = 4
CHUNK = SEQ_PER_CHUNK * MAX_LEN
NCHUNK = ROWS_PER_W // CHUNK
DV = EMBED_DIM // L


def _body(x_hbm, tok_hbm, pos_hbm, out_hbm, pos_v, idx_v, rows_v, gsem):
    wid = lax.axis_index("s") * NC + lax.axis_index("c")
    base = wid * ROWS_PER_W
    pltpu.sync_copy(pos_hbm, pos_v)

    @pl.loop(0, NCHUNK)
    def _chunk(g):
        start = base + g * CHUNK
        pltpu.sync_copy(x_hbm.at[pl.ds(start, CHUNK)], idx_v.at[pl.ds(0, CHUNK)])
        pltpu.async_copy(tok_hbm.at[idx_v.at[pl.ds(0, CHUNK)]], rows_v.at[0], gsem).wait()

        @pl.loop(0, MAX_LEN)
        def _pos(l):
            for d in range(DV):
                pv = pos_v[l, pl.ds(d * L, L)]
                for s in range(SEQ_PER_CHUNK):
                    r = s * MAX_LEN + l
                    rows_v[0, r, pl.ds(d * L, L)] += pv

        pltpu.sync_copy(rows_v.at[0], out_hbm.at[pl.ds(start, CHUNK)])


def kernel(x, token_table, pos_table):
    xf = x.reshape(ROWS).astype(jnp.int32)
    out = pl.kernel(
        _body,
        out_type=jax.ShapeDtypeStruct((ROWS, EMBED_DIM), jnp.float32),
        mesh=plsc.VectorSubcoreMesh(core_axis_name="c", subcore_axis_name="s"),
        compiler_params=pltpu.CompilerParams(use_tc_tiling_on_sc=False),
        scratch_types=[
            pltpu.VMEM((MAX_LEN, EMBED_DIM), jnp.float32),
            pltpu.VMEM((2 * CHUNK,), jnp.int32),
            pltpu.VMEM((2, CHUNK, EMBED_DIM), jnp.float32),
            pltpu.SemaphoreType.DMA,
        ],
    )(xf, token_table, pos_table)
    return out.reshape(BATCH, MAX_LEN, EMBED_DIM)


# speedup vs baseline: 1.0766x; 1.0766x over previous
import jax
import jax.numpy as jnp
from jax import lax
from jax.experimental import pallas as pl
from jax.experimental.pallas import tpu as pltpu
from jax.experimental.pallas import tpu_sc as plsc

VOCAB = 1_000_000
MAX_LEN = 200
EMBED_DIM = 64
BATCH = 4096

NC, NS, L = 2, 16, 16
NW = NC * NS
ROWS = BATCH * MAX_LEN
ROWS_PER_W = ROWS // NW
CHUNK = MAX_LEN
NCHUNK = ROWS_PER_W // CHUNK
NBUF = 4
DV = EMBED_DIM // L


def _body(x_hbm, tok_hbm, pos_hbm, out_hbm, pos_v, idx_v, rows_v, gsems, wsems):
    wid = lax.axis_index("s") * NC + lax.axis_index("c")
    base = wid * ROWS_PER_W
    pltpu.sync_copy(pos_hbm, pos_v)
    pltpu.sync_copy(x_hbm.at[pl.ds(base, ROWS_PER_W)], idx_v)

    def gather(G, b):
        return pltpu.make_async_copy(
            tok_hbm.at[idx_v.at[pl.ds(G * CHUNK, CHUNK)]], rows_v.at[b], gsems[b])

    def write(G, b):
        return pltpu.make_async_copy(
            rows_v.at[b], out_hbm.at[pl.ds(base + G * CHUNK, CHUNK)], wsems[b])

    gather(0, 0).start()
    gather(1, 1).start()

    @pl.loop(0, NCHUNK, step=NBUF)
    def _ring(g):
        for b in range(NBUF):
            G = g + b
            nb = (b + 2) % NBUF

            @pl.when(G + 2 < NCHUNK)
            def _prefetch():
                @pl.when(G >= 2)
                def _drain():
                    write(G - 2, nb).wait()
                gather(G + 2, nb).start()

            gather(G, b).wait()

            @plsc.parallel_loop(0, CHUNK, unroll=4)
            def _pos(r):
                for d in range(DV):
                    plsc.addupdate(rows_v.at[b, r, pl.ds(d * L, L)],
                                   pos_v[r, pl.ds(d * L, L)])

            write(G, b).start()

    for b in range(NBUF):
        write(NCHUNK - NBUF + b, b).wait()


def kernel(x, token_table, pos_table):
    xf = x.reshape(ROWS).astype(jnp.int32)
    out = pl.kernel(
        _body,
        out_type=jax.ShapeDtypeStruct((ROWS, EMBED_DIM), jnp.float32),
        mesh=plsc.VectorSubcoreMesh(core_axis_name="c", subcore_axis_name="s"),
        compiler_params=pltpu.CompilerParams(use_tc_tiling_on_sc=False),
        scratch_types=[
            pltpu.VMEM((MAX_LEN, EMBED_DIM), jnp.float32),
            pltpu.VMEM((ROWS_PER_W,), jnp.int32),
            pltpu.VMEM((NBUF, CHUNK, EMBED_DIM), jnp.float32),
            [pltpu.SemaphoreType.DMA] * NBUF,
            [pltpu.SemaphoreType.DMA] * NBUF,
        ],
    )(xf, token_table, pos_table)
    return out.reshape(BATCH, MAX_LEN, EMBED_DIM)
